# Initial kernel scaffold; baseline (speedup 1.0000x reference)
import jax
import jax.numpy as jnp
from jax.experimental import pallas as pl


def kernel(x, pe, batch, W_enc, b_enc, W1, b1, W2, b2):
    raise NotImplementedError("write your pallas kernel here")


# speedup vs baseline: 4.6279x; 4.6279x over previous
import jax
import jax.numpy as jnp
from jax.experimental import pallas as pl
from jax.experimental.pallas import tpu as pltpu

N = 100000
D_X = 128
PE_DIM = 8
HID = 128
OUT = 16
G = 512
B = 2000
NB = N // B


def _tc_kernel(xb, peb, bb, W_enc, b_enc, W1, b1, W2, b2, out_ref,
               accx, accp):
    step = pl.program_id(0)

    @pl.when(step == 0)
    def _init():
        accx[...] = jnp.zeros_like(accx)
        accp[...] = jnp.zeros_like(accp)

    ids = bb[0, 0, :]
    seg = jax.lax.broadcasted_iota(jnp.int32, (1, G), 1)
    onehot = (ids[:, None] == seg).astype(jnp.float32)

    accx[...] += jax.lax.dot_general(
        onehot, xb[...], (((0,), (0,)), ((), ())),
        preferred_element_type=jnp.float32)
    pe1 = jnp.concatenate(
        [peb[...], jnp.ones((B, 1), jnp.float32)], axis=1)
    accp[...] += jax.lax.dot_general(
        onehot, pe1, (((0,), (0,)), ((), ())),
        preferred_element_type=jnp.float32)

    @pl.when(step == NB - 1)
    def _finish():
        cnt = jnp.maximum(accp[:, PE_DIM:PE_DIM + 1], 1.0)
        pooled_x = accx[...] / cnt
        pooled_pe = accp[:, :PE_DIM] / cnt
        h = (jax.lax.dot_general(pooled_x, W_enc[:, :D_X],
                                 (((1,), (1,)), ((), ())),
                                 preferred_element_type=jnp.float32)
             + jax.lax.dot_general(pooled_pe, W_enc[:, D_X:],
                                   (((1,), (1,)), ((), ())),
                                   preferred_element_type=jnp.float32)
             + b_enc[...])
        h1 = jnp.maximum(
            jax.lax.dot_general(h, W1[...], (((1,), (1,)), ((), ())),
                                preferred_element_type=jnp.float32)
            + b1[...], 0.0)
        out_ref[...] = (
            jax.lax.dot_general(h1, W2[...], (((1,), (1,)), ((), ())),
                                preferred_element_type=jnp.float32)
            + b2[...])


def kernel(x, pe, batch, W_enc, b_enc, W1, b1, W2, b2):
    batch3 = batch.astype(jnp.int32).reshape(NB, 1, B)
    grid = (NB,)
    const = lambda i: (0, 0)
    out = pl.pallas_call(
        _tc_kernel,
        grid=grid,
        in_specs=[
            pl.BlockSpec((B, D_X), lambda i: (i, 0)),
            pl.BlockSpec((B, PE_DIM), lambda i: (i, 0)),
            pl.BlockSpec((1, 1, B), lambda i: (i, 0, 0)),
            pl.BlockSpec((HID, D_X + PE_DIM), const),
            pl.BlockSpec((1, HID), const),
            pl.BlockSpec((HID, HID), const),
            pl.BlockSpec((1, HID), const),
            pl.BlockSpec((OUT, HID), const),
            pl.BlockSpec((1, OUT), const),
        ],
        out_specs=pl.BlockSpec((G, OUT), const),
        out_shape=jax.ShapeDtypeStruct((G, OUT), jnp.float32),
        scratch_shapes=[
            pltpu.VMEM((G, D_X), jnp.float32),
            pltpu.VMEM((G, PE_DIM + 1), jnp.float32),
        ],
    )(x, pe, batch3, W_enc, b_enc.reshape(1, HID), W1,
      b1.reshape(1, HID), W2, b2.reshape(1, OUT))
    return out
